# baseline (speedup 1.0000x reference)
import functools

import jax
import jax.numpy as jnp
from jax import lax
from jax.experimental import pallas as pl
from jax.experimental.pallas import tpu as pltpu
from jax.experimental.pallas import tpu_sc as plsc

B, S, H = 1024, 200, 128
N = B * S
NUM_WORKERS = 32
ROWS_PER_W = N // NUM_WORKERS
CHUNK = 64
N_CHUNKS = ROWS_PER_W // CHUNK
GROUP = 4
NGROUPS = N_CHUNKS // GROUP
GBUF = 3

_mesh = plsc.VectorSubcoreMesh(core_axis_name="c", subcore_axis_name="s")


@functools.partial(
    pl.kernel,
    mesh=_mesh,
    out_type=jax.ShapeDtypeStruct((N, H), jnp.float32),
    scratch_types=(
        [pltpu.VMEM((N_CHUNKS, CHUNK), jnp.int32)]
        + [pltpu.VMEM((GROUP * CHUNK, H), jnp.float32) for _ in range(GBUF)]
        + [pltpu.SemaphoreType.DMA for _ in range(2 * GBUF)]
    ),
)
def _gather_kernel(idx_hbm, table_hbm, out_hbm, idx_v, *rest):
    bufs = rest[:GBUF]
    gs = rest[GBUF:2 * GBUF]
    ws = rest[2 * GBUF:]
    wid = lax.axis_index("s") * 2 + lax.axis_index("c")
    base = wid * ROWS_PER_W
    pltpu.sync_copy(idx_hbm.at[wid], idx_v)

    def gather_descs(g, hb):
        return [
            pltpu.make_async_copy(
                table_hbm.at[idx_v.at[g * GROUP + j]],
                bufs[hb].at[pl.ds(j * CHUNK, CHUNK)],
                gs[hb])
            for j in range(GROUP)
        ]

    def write_desc(g, hb):
        return pltpu.make_async_copy(
            bufs[hb],
            out_hbm.at[pl.ds(base + g * GROUP * CHUNK, GROUP * CHUNK)],
            ws[hb])

    for hb in range(GBUF):
        for d in gather_descs(hb, hb):
            d.start()

    def body(i, _):
        gbase = i * GBUF
        for hb in range(GBUF):
            g = gbase + hb
            for d in gather_descs(g, hb):
                d.wait()
            write_desc(g, hb).start()
        for hb in range(GBUF):
            g = gbase + hb + GBUF

            @pl.when(g < NGROUPS)
            def _(g=g, hb=hb):
                write_desc(g - GBUF, hb).wait()
                for d in gather_descs(g, hb):
                    d.start()

        return ()

    n_iters = NGROUPS // GBUF
    lax.fori_loop(0, n_iters, body, (), unroll=False)

    for g in range(n_iters * GBUF, NGROUPS):
        hb = g % GBUF
        for d in gather_descs(g, hb):
            d.wait()
        write_desc(g, hb).start()

    for g in range(NGROUPS - GBUF, NGROUPS):
        write_desc(g, g % GBUF).wait()


def kernel(x, embedding):
    idx = x.reshape(NUM_WORKERS, N_CHUNKS, CHUNK)
    out = _gather_kernel(idx, embedding)
    return out.reshape(B, S, H)


# speedup vs baseline: 1.0555x; 1.0555x over previous
import functools

import jax
import jax.numpy as jnp
from jax import lax
from jax.experimental import pallas as pl
from jax.experimental.pallas import tpu as pltpu
from jax.experimental.pallas import tpu_sc as plsc

B, S, H = 1024, 200, 128
N = B * S
NUM_WORKERS = 32
ROWS_PER_W = N // NUM_WORKERS
CHUNK = 64
N_CHUNKS = ROWS_PER_W // CHUNK
NBUF = 10

_mesh = plsc.VectorSubcoreMesh(core_axis_name="c", subcore_axis_name="s")


@functools.partial(
    pl.kernel,
    mesh=_mesh,
    out_type=jax.ShapeDtypeStruct((N, H), jnp.float32),
    scratch_types=(
        [pltpu.VMEM((N_CHUNKS, CHUNK), jnp.int32)]
        + [pltpu.VMEM((CHUNK, H), jnp.float32) for _ in range(NBUF)]
        + [pltpu.SemaphoreType.DMA for _ in range(2 * NBUF)]
    ),
)
def _gather_kernel(idx_hbm, table_hbm, out_hbm, idx_v, *rest):
    bufs = rest[:NBUF]
    gs = rest[NBUF:2 * NBUF]
    ws = rest[2 * NBUF:]
    wid = lax.axis_index("s") * 2 + lax.axis_index("c")
    base = wid * ROWS_PER_W
    pltpu.sync_copy(idx_hbm.at[wid], idx_v)

    def gather_desc(c, buf, sem):
        return pltpu.make_async_copy(table_hbm.at[idx_v.at[c]], buf, sem)

    def write_desc(c, buf, sem):
        return pltpu.make_async_copy(
            buf, out_hbm.at[pl.ds(base + c * CHUNK, CHUNK)], sem)

    for b in range(NBUF):
        gather_desc(b, bufs[b], gs[b]).start()

    def body(i, _):
        cbase = i * NBUF
        for b in range(NBUF):
            c = cbase + b
            gather_desc(c, bufs[b], gs[b]).wait()
            write_desc(c, bufs[b], ws[b]).start()
        for b in range(NBUF):
            c = cbase + b + NBUF

            @pl.when(c < N_CHUNKS)
            def _(c=c, b=b):
                write_desc(c - NBUF, bufs[b], ws[b]).wait()
                gather_desc(c, bufs[b], gs[b]).start()

        return ()

    lax.fori_loop(0, N_CHUNKS // NBUF, body, (), unroll=False)

    cL = N_CHUNKS - NBUF
    for b in range(NBUF):
        write_desc(cL + b, bufs[b], ws[b]).wait()


def kernel(x, embedding):
    idx = x.reshape(NUM_WORKERS, N_CHUNKS, CHUNK)
    out = _gather_kernel(idx, embedding)
    return out.reshape(B, S, H)
